# baseline (speedup 1.0000x reference)
import functools
import math

import jax
import jax.numpy as jnp
from jax import lax
from jax.experimental import pallas as pl
from jax.experimental.pallas import tpu as pltpu
from jax.experimental.pallas import tpu_sc as plsc

D_MODEL = 64
SCALE = math.sqrt(D_MODEL)

_info = plsc.get_sparse_core_info()
NC = _info.num_cores
NS = _info.num_subcores
NW = NC * NS
L = _info.num_lanes

CH = 128
B_TOTAL = 4096 * 200
PER_W = B_TOTAL // NW
NCHUNK = PER_W // CH


def _body(x_ref, table_ref, out_ref, idx_v, rows_v, gsem):
    wid = lax.axis_index("s") * NC + lax.axis_index("c")
    pltpu.sync_copy(x_ref.at[wid], idx_v)

    def chunk(j, carry):
        pltpu.async_copy(table_ref.at[idx_v.at[j]], rows_v, gsem).wait()

        def rowf(r, c2):
            for c in range(D_MODEL // L):
                rows_v[r, pl.ds(c * L, L)] = rows_v[r, pl.ds(c * L, L)] * SCALE
            return c2

        lax.fori_loop(0, CH, rowf, 0)
        pltpu.sync_copy(rows_v, out_ref.at[wid, j])
        return carry

    lax.fori_loop(0, NCHUNK, chunk, 0)


@functools.partial(
    pl.kernel,
    mesh=plsc.VectorSubcoreMesh(core_axis_name="c", subcore_axis_name="s"),
    out_type=jax.ShapeDtypeStruct((NW, NCHUNK, CH, D_MODEL), jnp.float32),
    scratch_types=[
        pltpu.VMEM((NCHUNK, CH), jnp.int32),
        pltpu.VMEM((CH, D_MODEL), jnp.float32),
        pltpu.SemaphoreType.DMA,
    ],
    compiler_params=pltpu.CompilerParams(use_tc_tiling_on_sc=False),
)
def _emb_lookup(x_ref, table_ref, out_ref, idx_v, rows_v, gsem):
    _body(x_ref, table_ref, out_ref, idx_v, rows_v, gsem)


def kernel(x, table):
    b, s = x.shape
    x32 = x.astype(jnp.int32).reshape(NW, NCHUNK, CH)
    out = _emb_lookup(x32, table)
    return out.reshape(b, s, D_MODEL)


# speedup vs baseline: 1.2081x; 1.2081x over previous
import functools
import math

import jax
import jax.numpy as jnp
from jax import lax
from jax.experimental import pallas as pl
from jax.experimental.pallas import tpu as pltpu
from jax.experimental.pallas import tpu_sc as plsc

D_MODEL = 64
SCALE = math.sqrt(D_MODEL)

_info = plsc.get_sparse_core_info()
NC = _info.num_cores
NS = _info.num_subcores
NW = NC * NS
L = _info.num_lanes

CH = 128
B_TOTAL = 4096 * 200
PER_W = B_TOTAL // NW
NCHUNK = PER_W // CH
NBUF = 4
NSUP = NCHUNK // NBUF
UN = 8


def _body(x_ref, table_ref, out_ref, idx_v, gbuf, sbuf, gsem, osem):
    wid = lax.axis_index("s") * NC + lax.axis_index("c")
    pltpu.sync_copy(x_ref.at[wid], idx_v)

    def start_gather(j, b):
        pltpu.async_copy(table_ref.at[idx_v.at[j]], gbuf.at[b], gsem.at[b])

    for b in range(NBUF):
        start_gather(b, b)

    def super_it(s, carry):
        for b in range(NBUF):
            j = s * NBUF + b
            pltpu.make_async_copy(
                table_ref.at[idx_v.at[j]], gbuf.at[b], gsem.at[b]
            ).wait()
            @pl.when(s > 0)
            def _():
                pltpu.make_async_copy(
                    sbuf.at[b], out_ref.at[wid, j], osem.at[b]
                ).wait()

            def mulf(rr, c2):
                for u in range(UN):
                    r = rr * UN + u
                    for c in range(D_MODEL // L):
                        sl = pl.ds(c * L, L)
                        sbuf[b, r, sl] = gbuf[b, r, sl] * SCALE
                return c2

            lax.fori_loop(0, CH // UN, mulf, 0)
            pltpu.async_copy(sbuf.at[b], out_ref.at[wid, j], osem.at[b])

            @pl.when(s < NSUP - 1)
            def _():
                start_gather(j + NBUF, b)

        return carry

    lax.fori_loop(0, NSUP, super_it, 0)

    for b in range(NBUF):
        j = NCHUNK - NBUF + b
        pltpu.make_async_copy(sbuf.at[b], out_ref.at[wid, j], osem.at[b]).wait()


@functools.partial(
    pl.kernel,
    mesh=plsc.VectorSubcoreMesh(core_axis_name="c", subcore_axis_name="s"),
    out_type=jax.ShapeDtypeStruct((NW, NCHUNK, CH, D_MODEL), jnp.float32),
    scratch_types=[
        pltpu.VMEM((NCHUNK, CH), jnp.int32),
        pltpu.VMEM((NBUF, CH, D_MODEL), jnp.float32),
        pltpu.VMEM((NBUF, CH, D_MODEL), jnp.float32),
        pltpu.SemaphoreType.DMA((NBUF,)),
        pltpu.SemaphoreType.DMA((NBUF,)),
    ],
    compiler_params=pltpu.CompilerParams(use_tc_tiling_on_sc=False),
)
def _emb_lookup(x_ref, table_ref, out_ref, idx_v, gbuf, sbuf, gsem, osem):
    _body(x_ref, table_ref, out_ref, idx_v, gbuf, sbuf, gsem, osem)


def kernel(x, table):
    b, s = x.shape
    x32 = x.astype(jnp.int32).reshape(NW, NCHUNK, CH)
    out = _emb_lookup(x32, table)
    return out.reshape(b, s, D_MODEL)


# speedup vs baseline: 1.2094x; 1.0011x over previous
import functools
import math

import jax
import jax.numpy as jnp
from jax import lax
from jax.experimental import pallas as pl
from jax.experimental.pallas import tpu as pltpu
from jax.experimental.pallas import tpu_sc as plsc

D_MODEL = 64
SCALE = math.sqrt(D_MODEL)

_info = plsc.get_sparse_core_info()
NC = _info.num_cores
NS = _info.num_subcores
NW = NC * NS
L = _info.num_lanes

ROWS = 4096
SEQ = 200
XR = ROWS // NW
G0 = 128
G1 = SEQ - G0
NG = 4
NSTB = 2
NSUP = XR // NG
UN = 10


def _body(x_ref, table_ref, out_ref, idx_v, gbuf, sbuf, gsem, osem):
    wid = lax.axis_index("s") * NC + lax.axis_index("c")
    base = wid * XR
    pltpu.sync_copy(x_ref.at[pl.ds(base, XR)], idx_v)

    def gather_pair(j, b):
        return (
            pltpu.make_async_copy(
                table_ref.at[idx_v.at[j, pl.ds(0, G0)]],
                gbuf.at[b, pl.ds(0, G0)],
                gsem.at[b],
            ),
            pltpu.make_async_copy(
                table_ref.at[idx_v.at[j, pl.ds(G0, G1)]],
                gbuf.at[b, pl.ds(G0, G1)],
                gsem.at[b],
            ),
        )

    def start_gather(j, b):
        c0, c1 = gather_pair(j, b)
        c0.start()
        c1.start()

    def wait_gather(j, b):
        c0, c1 = gather_pair(j, b)
        c0.wait()
        c1.wait()

    for b in range(NG):
        start_gather(b, b)

    def super_it(s, carry):
        for b in range(NG):
            j = s * NG + b
            b2 = b % NSTB
            dst = out_ref.at[base + j]
            wait_gather(j, b)

            def wait_store():
                pltpu.make_async_copy(sbuf.at[b2], dst, osem.at[b2]).wait()

            if b >= NSTB:
                wait_store()
            else:
                pl.when(s > 0)(wait_store)

            def mulf(rr, c2):
                for u in range(UN):
                    r = rr * UN + u
                    for c in range(D_MODEL // L):
                        sl = pl.ds(c * L, L)
                        sbuf[b2, r, sl] = gbuf[b, r, sl] * SCALE
                return c2

            lax.fori_loop(0, SEQ // UN, mulf, 0)
            pltpu.async_copy(sbuf.at[b2], dst, osem.at[b2])

            @pl.when(s < NSUP - 1)
            def _():
                start_gather(j + NG, b)

        return carry

    lax.fori_loop(0, NSUP, super_it, 0)

    for b2 in range(NSTB):
        j = XR - NSTB + b2
        pltpu.make_async_copy(
            sbuf.at[(j % NG) % NSTB], out_ref.at[base + j], osem.at[b2]
        ).wait()


@functools.partial(
    pl.kernel,
    mesh=plsc.VectorSubcoreMesh(core_axis_name="c", subcore_axis_name="s"),
    out_type=jax.ShapeDtypeStruct((ROWS, SEQ, D_MODEL), jnp.float32),
    scratch_types=[
        pltpu.VMEM((XR, SEQ), jnp.int32),
        pltpu.VMEM((NG, SEQ, D_MODEL), jnp.float32),
        pltpu.VMEM((NSTB, SEQ, D_MODEL), jnp.float32),
        pltpu.SemaphoreType.DMA((NG,)),
        pltpu.SemaphoreType.DMA((NSTB,)),
    ],
    compiler_params=pltpu.CompilerParams(use_tc_tiling_on_sc=False),
)
def _emb_lookup(x_ref, table_ref, out_ref, idx_v, gbuf, sbuf, gsem, osem):
    _body(x_ref, table_ref, out_ref, idx_v, gbuf, sbuf, gsem, osem)


def kernel(x, table):
    return _emb_lookup(x.astype(jnp.int32), table)
